# Initial kernel scaffold; baseline (speedup 1.0000x reference)
import jax
import jax.numpy as jnp
from jax.experimental import pallas as pl


def kernel(x):
    raise NotImplementedError("write your pallas kernel here")


# speedup vs baseline: 3.4136x; 3.4136x over previous
import jax
import jax.numpy as jnp
from jax import lax
from jax.experimental import pallas as pl

_B = 32
_F = 3
_P = 1024
_R = _B * _F
_N2 = int(0.9 * _P)


def _threefry2x32(ks0, ks1, x0, x1):
    ks2 = ks0 ^ ks1 ^ jnp.uint32(0x1BD11BDA)
    ks = (ks0, ks1, ks2)
    rots = ((13, 15, 26, 6), (17, 29, 16, 24))
    x0 = x0 + ks0
    x1 = x1 + ks1
    for g in range(5):
        for r in rots[g % 2]:
            x0 = x0 + x1
            x1 = (x1 << jnp.uint32(r)) | (x1 >> jnp.uint32(32 - r))
            x1 = x1 ^ x0
        x0 = x0 + ks[(g + 1) % 3]
        x1 = x1 + ks[(g + 2) % 3] + jnp.uint32(g + 1)
    return x0, x1


def _select_rank(keys, nbits, rem, cand):
    pref = jnp.zeros_like(rem)
    for bit in range(nbits - 1, -1, -1):
        ms = keys >> bit
        match = (ms >> 1) == pref
        if cand is not None:
            match = match & cand
        in0 = match & ((ms & 1) == 0)
        c0 = jnp.sum(in0.astype(jnp.int32), axis=1, keepdims=True)
        go1 = rem > c0
        pref = (pref << 1) | go1.astype(jnp.int32)
        rem = rem - jnp.where(go1, c0, 0)
    return pref, rem


def _mask_body(out_ref):
    col8 = lax.broadcasted_iota(jnp.uint32, (8, 128), 1)
    row8 = lax.broadcasted_iota(jnp.uint32, (8, 128), 0)
    s0, s1 = _threefry2x32(jnp.uint32(0), jnp.uint32(42),
                           jnp.zeros((8, 128), jnp.uint32), col8)
    top = row8 == jnp.uint32(0)
    sel_a = top & (col8 == jnp.uint32(0))
    sel_b = top & (col8 == jnp.uint32(1))
    def _pick(sel, v):
        vi = lax.bitcast_convert_type(v, jnp.int32)
        s = jnp.sum(jnp.where(sel, vi, 0))
        return lax.bitcast_convert_type(s, jnp.uint32)

    k1h = _pick(sel_a, s0)
    k1l = _pick(sel_a, s1)
    k2h = _pick(sel_b, s0)
    k2l = _pick(sel_b, s1)

    b_of_row = lax.broadcasted_iota(jnp.uint32, (_B, _F, 128), 0).reshape(_R, 128)
    f_of_row = lax.broadcasted_iota(jnp.int32, (_B, _F, 128), 1).reshape(_R, 128)
    u0, u1 = _threefry2x32(k1h, k1l, jnp.zeros((_R, 128), jnp.uint32), b_of_row)
    n1 = ((u0 ^ u1) >> jnp.uint32(22)).astype(jnp.int32)
    n_all = jnp.where(f_of_row == 0, n1,
                      jnp.where(f_of_row == 1, _N2, _P - n1))
    n = n_all[:, :1]

    r_i = lax.broadcasted_iota(jnp.uint32, (_R, _P), 0)
    j_i = lax.broadcasted_iota(jnp.uint32, (_R, _P), 1)
    cnt = r_i * jnp.uint32(_P) + j_i
    y0, y1 = _threefry2x32(k2h, k2l, jnp.zeros((_R, _P), jnp.uint32), cnt)
    m = ((y0 ^ y1) >> jnp.uint32(9)).astype(jnp.int32)

    t, rem = _select_rank(m, 23, n, None)
    eq = m == t
    jj = lax.broadcasted_iota(jnp.int32, (_R, _P), 1)
    j_thresh, _ = _select_rank(jj, 10, rem, eq)
    mask = (m < t) | (eq & (jj <= j_thresh) & (n > 0))
    out_ref[...] = mask


def kernel(x):
    masks = pl.pallas_call(
        _mask_body,
        out_shape=jax.ShapeDtypeStruct((_R, _P), jnp.bool_),
    )()
    return masks.reshape(_B, _F * _P)
